# baseline (speedup 1.0000x reference)
import functools

import jax
import jax.numpy as jnp
from jax import lax
from jax.experimental import pallas as pl

_BETA = 0.25


def _body(x_ref, cb_ref, ids_ref, dec_ref, loss_ref, *, num_layers, block_n, k):
    i = pl.program_id(0)
    x0 = x_ref[...]
    resid = x0
    iota = lax.broadcasted_iota(jnp.int32, (block_n, k), 1)
    acc = jnp.float32(0.0)
    for layer in range(num_layers):
        cb = cb_ref[layer]
        norm = jnp.sqrt(jnp.sum(cb * cb, axis=1, keepdims=True))
        cbn = cb / jnp.maximum(norm, 1e-12)
        b2 = jnp.sum(cbn * cbn, axis=1)
        a2 = jnp.sum(resid * resid, axis=1, keepdims=True)
        ab = lax.dot_general(
            resid, cbn, (((1,), (1,)), ((), ())),
            precision=lax.Precision.DEFAULT,
            preferred_element_type=jnp.float32)
        d2 = a2 + b2[None, :] - 2.0 * ab
        minv = jnp.min(d2, axis=1, keepdims=True)
        idx = jnp.min(jnp.where(d2 == minv, iota, k), axis=1)
        ids_ref[layer, :] = idx
        onehot = (iota == idx[:, None]).astype(jnp.float32)
        q = lax.dot_general(
            onehot, cb, (((1,), (0,)), ((), ())),
            precision=lax.Precision.HIGHEST,
            preferred_element_type=jnp.float32)
        ste = resid + (q - resid)
        resid = resid - ste
        acc = acc + jnp.sum(resid * resid)
    dec_ref[...] = x0 - resid

    @pl.when(i == 0)
    def _():
        loss_ref[...] = jnp.zeros((1, 1), jnp.float32)

    loss_ref[...] = loss_ref[...] + acc


def kernel(x, codebooks):
    n, d = x.shape
    num_layers, k, _ = codebooks.shape
    block_n = min(n, 1024)
    assert n % block_n == 0
    grid = (n // block_n,)

    ids, dec, loss = pl.pallas_call(
        functools.partial(_body, num_layers=num_layers, block_n=block_n, k=k),
        grid=grid,
        in_specs=[
            pl.BlockSpec((block_n, d), lambda i: (i, 0)),
            pl.BlockSpec((num_layers, k, d), lambda i: (0, 0, 0)),
        ],
        out_specs=[
            pl.BlockSpec((num_layers, block_n), lambda i: (0, i)),
            pl.BlockSpec((block_n, d), lambda i: (i, 0)),
            pl.BlockSpec((1, 1), lambda i: (0, 0)),
        ],
        out_shape=[
            jax.ShapeDtypeStruct((num_layers, n), jnp.int32),
            jax.ShapeDtypeStruct((n, d), jnp.float32),
            jax.ShapeDtypeStruct((1, 1), jnp.float32),
        ],
    )(x, codebooks)

    scale = jnp.float32((1.0 + _BETA) / (n * d))
    return (ids.T, dec, (loss[0, 0] * scale).astype(jnp.float32))


# speedup vs baseline: 1.9675x; 1.9675x over previous
import functools

import jax
import jax.numpy as jnp
from jax import lax
from jax.experimental import pallas as pl

_BETA = 0.25


def _body(x_ref, cb_ref, ids_ref, dec_ref, loss_ref, *, num_layers, block_n, k):
    i = pl.program_id(0)
    x0 = x_ref[...]
    resid = x0
    iota = lax.broadcasted_iota(jnp.int32, (block_n, k), 1)
    acc = jnp.float32(0.0)
    for layer in range(num_layers):
        cb = cb_ref[layer]
        norm = jnp.sqrt(jnp.sum(cb * cb, axis=1, keepdims=True))
        cbn = cb / jnp.maximum(norm, 1e-12)
        b2 = jnp.sum(cbn * cbn, axis=1)
        a2 = jnp.sum(resid * resid, axis=1, keepdims=True)
        ab = lax.dot_general(
            resid, cbn, (((1,), (1,)), ((), ())),
            precision=lax.Precision.DEFAULT,
            preferred_element_type=jnp.float32)
        d2 = a2 + b2[None, :] - 2.0 * ab
        minv = jnp.min(d2, axis=1, keepdims=True)
        idx = jnp.min(jnp.where(d2 == minv, iota, k), axis=1)
        ids_ref[layer, :] = idx
        onehot = (iota == idx[:, None]).astype(jnp.bfloat16)
        cb_hi = cb.astype(jnp.bfloat16)
        cb_lo = (cb - cb_hi.astype(jnp.float32)).astype(jnp.bfloat16)
        q = lax.dot_general(
            onehot, cb_hi, (((1,), (0,)), ((), ())),
            preferred_element_type=jnp.float32)
        q = q + lax.dot_general(
            onehot, cb_lo, (((1,), (0,)), ((), ())),
            preferred_element_type=jnp.float32)
        ste = resid + (q - resid)
        resid = resid - ste
        acc = acc + jnp.sum(resid * resid)
    dec_ref[...] = x0 - resid

    @pl.when(i == 0)
    def _():
        loss_ref[...] = jnp.zeros((1, 1), jnp.float32)

    loss_ref[...] = loss_ref[...] + acc


def kernel(x, codebooks):
    n, d = x.shape
    num_layers, k, _ = codebooks.shape
    block_n = min(n, 1024)
    assert n % block_n == 0
    grid = (n // block_n,)

    ids, dec, loss = pl.pallas_call(
        functools.partial(_body, num_layers=num_layers, block_n=block_n, k=k),
        grid=grid,
        in_specs=[
            pl.BlockSpec((block_n, d), lambda i: (i, 0)),
            pl.BlockSpec((num_layers, k, d), lambda i: (0, 0, 0)),
        ],
        out_specs=[
            pl.BlockSpec((num_layers, block_n), lambda i: (0, i)),
            pl.BlockSpec((block_n, d), lambda i: (i, 0)),
            pl.BlockSpec((1, 1), lambda i: (0, 0)),
        ],
        out_shape=[
            jax.ShapeDtypeStruct((num_layers, n), jnp.int32),
            jax.ShapeDtypeStruct((n, d), jnp.float32),
            jax.ShapeDtypeStruct((1, 1), jnp.float32),
        ],
    )(x, codebooks)

    scale = jnp.float32((1.0 + _BETA) / (n * d))
    return (ids.T, dec, (loss[0, 0] * scale).astype(jnp.float32))


# speedup vs baseline: 2.0561x; 1.0450x over previous
import functools

import jax
import jax.numpy as jnp
from jax import lax
from jax.experimental import pallas as pl

_BETA = 0.25


def _prep_body(cb_ref, cbn_ref, hi_ref, lo_ref, *, num_layers):
    for layer in range(num_layers):
        cb = cb_ref[layer]
        norm = jnp.sqrt(jnp.sum(cb * cb, axis=1, keepdims=True))
        cbn = cb / jnp.maximum(norm, 1e-12)
        cbn_ref[layer] = cbn
        hi = cb.astype(jnp.bfloat16)
        hi_ref[layer] = hi
        lo_ref[layer] = (cb - hi.astype(jnp.float32)).astype(jnp.bfloat16)


def _body(x_ref, cbn_ref, hi_ref, lo_ref, ids_ref, dec_ref, loss_ref, *,
          num_layers, block_n, k):
    i = pl.program_id(0)
    x0 = x_ref[...]
    resid = x0
    iota = lax.broadcasted_iota(jnp.int32, (k, block_n), 0)
    acc = jnp.float32(0.0)
    for layer in range(num_layers):
        cbn = cbn_ref[layer]
        b2 = jnp.sum(cbn * cbn, axis=1, keepdims=True)
        a2 = jnp.sum(resid * resid, axis=1, keepdims=True)
        a2t = jnp.transpose(a2, (1, 0))
        s = lax.dot_general(
            cbn, resid, (((1,), (1,)), ((), ())),
            precision=lax.Precision.DEFAULT,
            preferred_element_type=jnp.float32)
        d2 = (a2t + b2) - 2.0 * s
        minv = jnp.min(d2, axis=0, keepdims=True)
        idx = jnp.min(jnp.where(d2 == minv, iota, k), axis=0)
        ids_ref[layer, :] = idx
        onehot = (iota == idx[None, :]).astype(jnp.bfloat16)
        q = lax.dot_general(
            onehot, hi_ref[layer], (((0,), (0,)), ((), ())),
            preferred_element_type=jnp.float32)
        q = q + lax.dot_general(
            onehot, lo_ref[layer], (((0,), (0,)), ((), ())),
            preferred_element_type=jnp.float32)
        ste = resid + (q - resid)
        resid = resid - ste
        acc = acc + jnp.sum(resid * resid)
    dec_ref[...] = x0 - resid

    @pl.when(i == 0)
    def _():
        loss_ref[...] = jnp.zeros((1, 1), jnp.float32)

    loss_ref[...] = loss_ref[...] + acc


def kernel(x, codebooks):
    n, d = x.shape
    num_layers, k, _ = codebooks.shape
    block_n = min(n, 1024)
    assert n % block_n == 0

    cbn, cb_hi, cb_lo = pl.pallas_call(
        functools.partial(_prep_body, num_layers=num_layers),
        out_shape=[
            jax.ShapeDtypeStruct((num_layers, k, d), jnp.float32),
            jax.ShapeDtypeStruct((num_layers, k, d), jnp.bfloat16),
            jax.ShapeDtypeStruct((num_layers, k, d), jnp.bfloat16),
        ],
    )(codebooks)

    whole = pl.BlockSpec((num_layers, k, d), lambda i: (0, 0, 0))
    ids, dec, loss = pl.pallas_call(
        functools.partial(_body, num_layers=num_layers, block_n=block_n, k=k),
        grid=(n // block_n,),
        in_specs=[
            pl.BlockSpec((block_n, d), lambda i: (i, 0)),
            whole, whole, whole,
        ],
        out_specs=[
            pl.BlockSpec((num_layers, block_n), lambda i: (0, i)),
            pl.BlockSpec((block_n, d), lambda i: (i, 0)),
            pl.BlockSpec((1, 1), lambda i: (0, 0)),
        ],
        out_shape=[
            jax.ShapeDtypeStruct((num_layers, n), jnp.int32),
            jax.ShapeDtypeStruct((n, d), jnp.float32),
            jax.ShapeDtypeStruct((1, 1), jnp.float32),
        ],
    )(x, cbn, cb_hi, cb_lo)

    scale = jnp.float32((1.0 + _BETA) / (n * d))
    return (ids.T, dec, (loss[0, 0] * scale).astype(jnp.float32))


# speedup vs baseline: 2.8068x; 1.3651x over previous
import functools

import jax
import jax.numpy as jnp
from jax import lax
from jax.experimental import pallas as pl

_BETA = 0.25


def _prep_body(cb_ref, cbn_ref, hi_ref, lo_ref, *, num_layers):
    for layer in range(num_layers):
        cb = cb_ref[layer]
        norm = jnp.sqrt(jnp.sum(cb * cb, axis=1, keepdims=True))
        cbn = cb / jnp.maximum(norm, 1e-12)
        cbn_ref[layer] = cbn
        hi = cb.astype(jnp.bfloat16)
        hi_ref[layer] = hi
        lo_ref[layer] = (cb - hi.astype(jnp.float32)).astype(jnp.bfloat16)


def _body(x_ref, cbn_ref, hi_ref, lo_ref, ids_ref, dec_ref, loss_ref, *,
          num_layers, block_n, k, chains):
    i = pl.program_id(0)
    h = block_n // chains
    iota = lax.broadcasted_iota(
        jnp.int32, (k, h), 0).astype(jnp.float32)
    kf = jnp.float32(k)
    resids = [x_ref[c * h:(c + 1) * h, :] for c in range(chains)]
    accs = [jnp.float32(0.0) for _ in range(chains)]
    for layer in range(num_layers):
        cbn = cbn_ref[layer]
        hi = hi_ref[layer]
        lo = lo_ref[layer]
        b2 = jnp.sum(cbn * cbn, axis=1, keepdims=True)
        for c in range(chains):
            resid = resids[c]
            a2 = jnp.sum(resid * resid, axis=1, keepdims=True)
            a2t = jnp.transpose(a2, (1, 0))
            s = lax.dot_general(
                cbn, resid, (((1,), (1,)), ((), ())),
                precision=lax.Precision.DEFAULT,
                preferred_element_type=jnp.float32)
            d2 = (a2t + b2) - 2.0 * s
            minv = jnp.min(d2, axis=0, keepdims=True)
            idxf = jnp.min(jnp.where(d2 == minv, iota, kf), axis=0)
            ids_ref[layer, c * h:(c + 1) * h] = idxf.astype(jnp.int32)
            onehot = (iota == idxf[None, :]).astype(jnp.bfloat16)
            q = lax.dot_general(
                onehot, hi, (((0,), (0,)), ((), ())),
                preferred_element_type=jnp.float32)
            q = q + lax.dot_general(
                onehot, lo, (((0,), (0,)), ((), ())),
                preferred_element_type=jnp.float32)
            ste = resid + (q - resid)
            resids[c] = resid - ste
            accs[c] = accs[c] + jnp.sum(resids[c] * resids[c])
    for c in range(chains):
        dec_ref[c * h:(c + 1) * h, :] = (
            x_ref[c * h:(c + 1) * h, :] - resids[c])
    acc = sum(accs)

    @pl.when(i == 0)
    def _():
        loss_ref[...] = jnp.zeros((1, 1), jnp.float32)

    loss_ref[...] = loss_ref[...] + acc


def kernel(x, codebooks):
    n, d = x.shape
    num_layers, k, _ = codebooks.shape
    block_n = min(n, 1024)
    assert n % block_n == 0

    cbn, cb_hi, cb_lo = pl.pallas_call(
        functools.partial(_prep_body, num_layers=num_layers),
        out_shape=[
            jax.ShapeDtypeStruct((num_layers, k, d), jnp.float32),
            jax.ShapeDtypeStruct((num_layers, k, d), jnp.bfloat16),
            jax.ShapeDtypeStruct((num_layers, k, d), jnp.bfloat16),
        ],
    )(codebooks)

    whole = pl.BlockSpec((num_layers, k, d), lambda i: (0, 0, 0))
    ids, dec, loss = pl.pallas_call(
        functools.partial(_body, num_layers=num_layers, block_n=block_n, k=k,
                          chains=2 if block_n % 2 == 0 else 1),
        grid=(n // block_n,),
        in_specs=[
            pl.BlockSpec((block_n, d), lambda i: (i, 0)),
            whole, whole, whole,
        ],
        out_specs=[
            pl.BlockSpec((num_layers, block_n), lambda i: (0, i)),
            pl.BlockSpec((block_n, d), lambda i: (i, 0)),
            pl.BlockSpec((1, 1), lambda i: (0, 0)),
        ],
        out_shape=[
            jax.ShapeDtypeStruct((num_layers, n), jnp.int32),
            jax.ShapeDtypeStruct((n, d), jnp.float32),
            jax.ShapeDtypeStruct((1, 1), jnp.float32),
        ],
    )(x, cbn, cb_hi, cb_lo)

    scale = jnp.float32((1.0 + _BETA) / (n * d))
    return (ids.T, dec, (loss[0, 0] * scale).astype(jnp.float32))
